# baseline (device time: 79332 ns/iter reference)
import jax
import jax.numpy as jnp
from jax import lax
from jax.experimental import pallas as pl
from jax.experimental.pallas import tpu as pltpu

ROWS = 4096
COLS = 1024
SUB = 8
CCOMM = 256
NCOMM = 8
CKEEP = 512
NKEEP = ROWS // CKEEP


def kernel(x, dest):
    order = jnp.argsort(dest, stable=True)
    xs = x.at[order].get(mode="promise_in_bounds", unique_indices=True)
    xs = xs.astype(jnp.bfloat16).reshape(ROWS * SUB, 128)
    c0 = jnp.sum(dest == 0).astype(jnp.int32).reshape(1)

    def body(xs_ref, c0_ref, out_ref,
             ysend_sems, yrecv_sems, xsend_sems, xrecv_sems, copy_sems):
        my_x = lax.axis_index("x")
        my_y = lax.axis_index("y")
        ypeer = (my_x, 1 - my_y)
        xpeer = (1 - my_x, my_y)

        barrier = pltpu.get_barrier_semaphore()
        for nbr in (ypeer, xpeer):
            pl.semaphore_signal(barrier, inc=1, device_id=nbr,
                                device_id_type=pl.DeviceIdType.MESH)
        pl.semaphore_wait(barrier, 2)

        c0_mine = c0_ref[0]
        is0 = my_y == 0
        K = jnp.where(is0, c0_mine, ROWS - c0_mine)
        S = ROWS - K
        src_keep = jnp.where(is0, 0, c0_mine)
        src_send = jnp.where(is0, c0_mine, 0)
        my_keep_off = jnp.where(is0, 0, S)
        my_recv_off = jnp.where(is0, K, 0)
        peer_dst_off = jnp.where(is0, 0, K)

        H0 = S // 2
        col0 = my_x == 0
        my_base = jnp.where(col0, 0, H0)
        my_len = jnp.where(col0, H0, S - H0)
        ot_base = jnp.where(col0, H0, 0)
        ot_len = jnp.where(col0, S - H0, H0)

        n_y = (my_len + CCOMM - 1) // CCOMM
        n_x = (ot_len + CCOMM - 1) // CCOMM
        n_keep = (K + CKEEP - 1) // CKEEP

        def chunk_start(k, total, csz):
            return jnp.maximum(0, jnp.minimum(k * csz, total - csz))

        def sl(ref, logical_off, csz):
            return ref.at[pl.ds(pl.multiple_of(logical_off * SUB, SUB),
                                csz * SUB)]

        def ysend_desc(k):
            s = my_base + chunk_start(k, my_len, CCOMM)
            return pltpu.make_async_remote_copy(
                src_ref=sl(xs_ref, src_send + s, CCOMM),
                dst_ref=sl(out_ref, peer_dst_off + s, CCOMM),
                send_sem=ysend_sems.at[k], recv_sem=yrecv_sems.at[k],
                device_id=ypeer, device_id_type=pl.DeviceIdType.MESH)

        def xfwd_desc(k):
            s = my_recv_off + my_base + chunk_start(k, my_len, CCOMM)
            return pltpu.make_async_remote_copy(
                src_ref=sl(out_ref, s, CCOMM),
                dst_ref=sl(out_ref, s, CCOMM),
                send_sem=xsend_sems.at[k], recv_sem=xrecv_sems.at[k],
                device_id=xpeer, device_id_type=pl.DeviceIdType.MESH)

        for k in range(NCOMM):
            @pl.when(k < n_y)
            def _(k=k):
                ysend_desc(k).start()

        def keep_desc(k):
            s = chunk_start(k, K, CKEEP)
            return pltpu.make_async_copy(
                sl(xs_ref, src_keep + s, CKEEP),
                sl(out_ref, my_keep_off + s, CKEEP),
                copy_sems.at[k])

        for k in range(NKEEP):
            @pl.when(k < n_keep)
            def _(k=k):
                keep_desc(k).start()

        for k in range(NCOMM):
            @pl.when(k < n_y)
            def _(k=k):
                ysend_desc(k).wait_recv()
                xfwd_desc(k).start()

        for k in range(NKEEP):
            @pl.when(k < n_keep)
            def _(k=k):
                keep_desc(k).wait()

        for k in range(NCOMM):
            @pl.when(k < n_x)
            def _(k=k):
                s = my_recv_off + ot_base + chunk_start(k, ot_len, CCOMM)
                pltpu.make_async_remote_copy(
                    src_ref=sl(xs_ref, 0, CCOMM),
                    dst_ref=sl(out_ref, s, CCOMM),
                    send_sem=xsend_sems.at[k], recv_sem=xrecv_sems.at[k],
                    device_id=xpeer,
                    device_id_type=pl.DeviceIdType.MESH).wait_recv()

        for k in range(NCOMM):
            @pl.when(k < n_y)
            def _(k=k):
                ysend_desc(k).wait_send()
                xfwd_desc(k).wait_send()

    out = pl.pallas_call(
        body,
        out_shape=jax.ShapeDtypeStruct((ROWS * SUB, 128), jnp.bfloat16),
        in_specs=[
            pl.BlockSpec(memory_space=pltpu.VMEM),
            pl.BlockSpec(memory_space=pltpu.SMEM),
        ],
        out_specs=pl.BlockSpec(memory_space=pltpu.VMEM),
        scratch_shapes=[
            pltpu.SemaphoreType.DMA((NCOMM,)),
            pltpu.SemaphoreType.DMA((NCOMM,)),
            pltpu.SemaphoreType.DMA((NCOMM,)),
            pltpu.SemaphoreType.DMA((NCOMM,)),
            pltpu.SemaphoreType.DMA((NKEEP,)),
        ],
        compiler_params=pltpu.CompilerParams(collective_id=0),
    )(xs, c0)
    return out.reshape(ROWS, COLS)


# device time: 78454 ns/iter; 1.0112x vs baseline; 1.0112x over previous
import jax
import jax.numpy as jnp
from jax import lax
from jax.experimental import pallas as pl
from jax.experimental.pallas import tpu as pltpu

ROWS = 4096
COLS = 1024
SUB = 8
CCOMM = 256
NCOMM = 8
CKEEP = 512
NKEEP = ROWS // CKEEP


def kernel(x, dest):
    iota = jnp.arange(ROWS, dtype=jnp.int32)
    order = jnp.sort((dest << 13) | iota) & 0x1FFF
    xs = x.at[order].get(mode="promise_in_bounds", unique_indices=True)
    xs = xs.astype(jnp.bfloat16).reshape(ROWS * SUB, 128)
    c0 = jnp.sum(dest == 0).astype(jnp.int32).reshape(1)

    def body(xs_ref, c0_ref, out_ref,
             ysend_sems, yrecv_sems, xsend_sems, xrecv_sems, copy_sems):
        my_x = lax.axis_index("x")
        my_y = lax.axis_index("y")
        ypeer = (my_x, 1 - my_y)
        xpeer = (1 - my_x, my_y)

        barrier = pltpu.get_barrier_semaphore()
        for nbr in (ypeer, xpeer):
            pl.semaphore_signal(barrier, inc=1, device_id=nbr,
                                device_id_type=pl.DeviceIdType.MESH)
        pl.semaphore_wait(barrier, 2)

        c0_mine = c0_ref[0]
        is0 = my_y == 0
        K = jnp.where(is0, c0_mine, ROWS - c0_mine)
        S = ROWS - K
        src_keep = jnp.where(is0, 0, c0_mine)
        src_send = jnp.where(is0, c0_mine, 0)
        my_keep_off = jnp.where(is0, 0, S)
        my_recv_off = jnp.where(is0, K, 0)
        peer_dst_off = jnp.where(is0, 0, K)

        H0 = S // 2
        col0 = my_x == 0
        my_base = jnp.where(col0, 0, H0)
        my_len = jnp.where(col0, H0, S - H0)
        ot_base = jnp.where(col0, H0, 0)
        ot_len = jnp.where(col0, S - H0, H0)

        n_y = (my_len + CCOMM - 1) // CCOMM
        n_x = (ot_len + CCOMM - 1) // CCOMM
        n_keep = (K + CKEEP - 1) // CKEEP

        def chunk_start(k, total, csz):
            return jnp.maximum(0, jnp.minimum(k * csz, total - csz))

        def sl(ref, logical_off, csz):
            return ref.at[pl.ds(pl.multiple_of(logical_off * SUB, SUB),
                                csz * SUB)]

        def ysend_desc(k):
            s = my_base + chunk_start(k, my_len, CCOMM)
            return pltpu.make_async_remote_copy(
                src_ref=sl(xs_ref, src_send + s, CCOMM),
                dst_ref=sl(out_ref, peer_dst_off + s, CCOMM),
                send_sem=ysend_sems.at[k], recv_sem=yrecv_sems.at[k],
                device_id=ypeer, device_id_type=pl.DeviceIdType.MESH)

        def xfwd_desc(k):
            s = my_recv_off + my_base + chunk_start(k, my_len, CCOMM)
            return pltpu.make_async_remote_copy(
                src_ref=sl(out_ref, s, CCOMM),
                dst_ref=sl(out_ref, s, CCOMM),
                send_sem=xsend_sems.at[k], recv_sem=xrecv_sems.at[k],
                device_id=xpeer, device_id_type=pl.DeviceIdType.MESH)

        for k in range(NCOMM):
            @pl.when(k < n_y)
            def _(k=k):
                ysend_desc(k).start()

        def keep_desc(k):
            s = chunk_start(k, K, CKEEP)
            return pltpu.make_async_copy(
                sl(xs_ref, src_keep + s, CKEEP),
                sl(out_ref, my_keep_off + s, CKEEP),
                copy_sems.at[k])

        for k in range(NKEEP):
            @pl.when(k < n_keep)
            def _(k=k):
                keep_desc(k).start()

        for k in range(NCOMM):
            @pl.when(k < n_y)
            def _(k=k):
                ysend_desc(k).wait_recv()
                xfwd_desc(k).start()

        for k in range(NKEEP):
            @pl.when(k < n_keep)
            def _(k=k):
                keep_desc(k).wait()

        for k in range(NCOMM):
            @pl.when(k < n_x)
            def _(k=k):
                s = my_recv_off + ot_base + chunk_start(k, ot_len, CCOMM)
                pltpu.make_async_remote_copy(
                    src_ref=sl(xs_ref, 0, CCOMM),
                    dst_ref=sl(out_ref, s, CCOMM),
                    send_sem=xsend_sems.at[k], recv_sem=xrecv_sems.at[k],
                    device_id=xpeer,
                    device_id_type=pl.DeviceIdType.MESH).wait_recv()

        for k in range(NCOMM):
            @pl.when(k < n_y)
            def _(k=k):
                ysend_desc(k).wait_send()
                xfwd_desc(k).wait_send()

    out = pl.pallas_call(
        body,
        out_shape=jax.ShapeDtypeStruct((ROWS * SUB, 128), jnp.bfloat16),
        in_specs=[
            pl.BlockSpec(memory_space=pltpu.VMEM),
            pl.BlockSpec(memory_space=pltpu.SMEM),
        ],
        out_specs=pl.BlockSpec(memory_space=pltpu.VMEM),
        scratch_shapes=[
            pltpu.SemaphoreType.DMA((NCOMM,)),
            pltpu.SemaphoreType.DMA((NCOMM,)),
            pltpu.SemaphoreType.DMA((NCOMM,)),
            pltpu.SemaphoreType.DMA((NCOMM,)),
            pltpu.SemaphoreType.DMA((NKEEP,)),
        ],
        compiler_params=pltpu.CompilerParams(collective_id=0),
    )(xs, c0)
    return out.reshape(ROWS, COLS)


# device time: 75958 ns/iter; 1.0444x vs baseline; 1.0329x over previous
import jax
import jax.numpy as jnp
from jax import lax
from jax.experimental import pallas as pl
from jax.experimental.pallas import tpu as pltpu

ROWS = 4096
COLS = 1024
SUB = 8
CCOMM = 128
NCOMM = 16
CKEEP = 512
NKEEP = ROWS // CKEEP


def kernel(x, dest):
    iota = jnp.arange(ROWS, dtype=jnp.int32)
    order = jnp.sort((dest << 13) | iota) & 0x1FFF
    xs = x.at[order].get(mode="promise_in_bounds", unique_indices=True)
    xs = xs.astype(jnp.bfloat16).reshape(ROWS * SUB, 128)
    c0 = jnp.sum(dest == 0).astype(jnp.int32).reshape(1)

    def body(xs_ref, c0_ref, out_ref,
             ysend_sems, yrecv_sems, xsend_sems, xrecv_sems, copy_sems):
        my_x = lax.axis_index("x")
        my_y = lax.axis_index("y")
        ypeer = (my_x, 1 - my_y)
        xpeer = (1 - my_x, my_y)

        barrier = pltpu.get_barrier_semaphore()
        for nbr in (ypeer, xpeer):
            pl.semaphore_signal(barrier, inc=1, device_id=nbr,
                                device_id_type=pl.DeviceIdType.MESH)
        pl.semaphore_wait(barrier, 2)

        c0_mine = c0_ref[0]
        is0 = my_y == 0
        K = jnp.where(is0, c0_mine, ROWS - c0_mine)
        S = ROWS - K
        src_keep = jnp.where(is0, 0, c0_mine)
        src_send = jnp.where(is0, c0_mine, 0)
        my_keep_off = jnp.where(is0, 0, S)
        my_recv_off = jnp.where(is0, K, 0)
        peer_dst_off = jnp.where(is0, 0, K)

        H0 = S // 2
        col0 = my_x == 0
        my_base = jnp.where(col0, 0, H0)
        my_len = jnp.where(col0, H0, S - H0)
        ot_base = jnp.where(col0, H0, 0)
        ot_len = jnp.where(col0, S - H0, H0)

        n_y = (my_len + CCOMM - 1) // CCOMM
        n_x = (ot_len + CCOMM - 1) // CCOMM
        n_keep = (K + CKEEP - 1) // CKEEP

        def chunk_start(k, total, csz):
            return jnp.maximum(0, jnp.minimum(k * csz, total - csz))

        def sl(ref, logical_off, csz):
            return ref.at[pl.ds(pl.multiple_of(logical_off * SUB, SUB),
                                csz * SUB)]

        def ysend_desc(k):
            s = my_base + chunk_start(k, my_len, CCOMM)
            return pltpu.make_async_remote_copy(
                src_ref=sl(xs_ref, src_send + s, CCOMM),
                dst_ref=sl(out_ref, peer_dst_off + s, CCOMM),
                send_sem=ysend_sems.at[k], recv_sem=yrecv_sems.at[k],
                device_id=ypeer, device_id_type=pl.DeviceIdType.MESH)

        def xfwd_desc(k):
            s = my_recv_off + my_base + chunk_start(k, my_len, CCOMM)
            return pltpu.make_async_remote_copy(
                src_ref=sl(out_ref, s, CCOMM),
                dst_ref=sl(out_ref, s, CCOMM),
                send_sem=xsend_sems.at[k], recv_sem=xrecv_sems.at[k],
                device_id=xpeer, device_id_type=pl.DeviceIdType.MESH)

        for k in range(NCOMM):
            @pl.when(k < n_y)
            def _(k=k):
                ysend_desc(k).start()

        def keep_desc(k):
            s = chunk_start(k, K, CKEEP)
            return pltpu.make_async_copy(
                sl(xs_ref, src_keep + s, CKEEP),
                sl(out_ref, my_keep_off + s, CKEEP),
                copy_sems.at[k])

        for k in range(NKEEP):
            @pl.when(k < n_keep)
            def _(k=k):
                keep_desc(k).start()

        for k in range(NCOMM):
            @pl.when(k < n_y)
            def _(k=k):
                ysend_desc(k).wait_recv()
                xfwd_desc(k).start()

        for k in range(NKEEP):
            @pl.when(k < n_keep)
            def _(k=k):
                keep_desc(k).wait()

        for k in range(NCOMM):
            @pl.when(k < n_x)
            def _(k=k):
                s = my_recv_off + ot_base + chunk_start(k, ot_len, CCOMM)
                pltpu.make_async_remote_copy(
                    src_ref=sl(xs_ref, 0, CCOMM),
                    dst_ref=sl(out_ref, s, CCOMM),
                    send_sem=xsend_sems.at[k], recv_sem=xrecv_sems.at[k],
                    device_id=xpeer,
                    device_id_type=pl.DeviceIdType.MESH).wait_recv()

        for k in range(NCOMM):
            @pl.when(k < n_y)
            def _(k=k):
                ysend_desc(k).wait_send()
                xfwd_desc(k).wait_send()

    out = pl.pallas_call(
        body,
        out_shape=jax.ShapeDtypeStruct((ROWS * SUB, 128), jnp.bfloat16),
        in_specs=[
            pl.BlockSpec(memory_space=pltpu.VMEM),
            pl.BlockSpec(memory_space=pltpu.SMEM),
        ],
        out_specs=pl.BlockSpec(memory_space=pltpu.VMEM),
        scratch_shapes=[
            pltpu.SemaphoreType.DMA((NCOMM,)),
            pltpu.SemaphoreType.DMA((NCOMM,)),
            pltpu.SemaphoreType.DMA((NCOMM,)),
            pltpu.SemaphoreType.DMA((NCOMM,)),
            pltpu.SemaphoreType.DMA((NKEEP,)),
        ],
        compiler_params=pltpu.CompilerParams(collective_id=0),
    )(xs, c0)
    return out.reshape(ROWS, COLS)
